# baseline (speedup 1.0000x reference)
import functools

import jax
import jax.numpy as jnp
from jax import lax
from jax.experimental import pallas as pl
from jax.experimental.pallas import tpu as pltpu

_N_CODES = 8192
_CODE_DIM = 32
_BETA = 0.25
_ROWS_PER_BLOCK = 1024
_CODE_CHUNK = 1024


def _vq_kernel(z_ref, z2_ref, e2_ref, cb_ref,
               zq_ref, codes_ref, counts_ref, loss_ref, perp_ref):
    i = pl.program_id(0)
    nprog = pl.num_programs(0)
    z = z_ref[...]
    z2 = z2_ref[...]
    e2 = e2_ref[...]
    rows = z.shape[0]

    run_min = jnp.full((rows, 1), jnp.inf, dtype=jnp.float32)
    run_idx = jnp.zeros((rows, 1), dtype=jnp.int32)
    arg_chunk = 2048
    lanes = lax.broadcasted_iota(jnp.int32, (rows, arg_chunk), 1)

    zb = z.astype(jnp.bfloat16)
    for j in range(_N_CODES // arg_chunk):
        cbj = cb_ref[j * arg_chunk:(j + 1) * arg_chunk, :]
        ze = lax.dot_general(zb, cbj.astype(jnp.bfloat16),
                             (((1,), (1,)), ((), ())),
                             preferred_element_type=jnp.float32)
        e2j = e2[:, j * arg_chunk:(j + 1) * arg_chunk]
        dist = (z2 + e2j) - 2.0 * ze
        cmin = jnp.min(dist, axis=1, keepdims=True)
        cidx = jnp.min(jnp.where(dist == cmin, lanes, jnp.int32(2**30)),
                       axis=1, keepdims=True) + j * arg_chunk
        upd = cmin < run_min
        run_idx = jnp.where(upd, cidx, run_idx)
        run_min = jnp.where(
            upd, cmin.astype(jnp.bfloat16).astype(jnp.float32), run_min)

    codes_ref[...] = run_idx

    zq = jnp.zeros((rows, _CODE_DIM), dtype=jnp.float32)
    lanes2 = lax.broadcasted_iota(jnp.int32, (rows, _CODE_CHUNK), 1)
    count_chunks = []
    for j in range(_N_CODES // _CODE_CHUNK):
        cbj = cb_ref[j * _CODE_CHUNK:(j + 1) * _CODE_CHUNK, :]
        onehot = (run_idx == lanes2 + j * _CODE_CHUNK).astype(jnp.bfloat16)
        cb_hi = cbj.astype(jnp.bfloat16)
        cb_lo = (cbj - cb_hi.astype(jnp.float32)).astype(jnp.bfloat16)
        zq = (zq
              + lax.dot_general(onehot, cb_hi, (((1,), (0,)), ((), ())),
                                preferred_element_type=jnp.float32)
              + lax.dot_general(onehot, cb_lo, (((1,), (0,)), ((), ())),
                                preferred_element_type=jnp.float32))
        count_chunks.append(
            jnp.sum(onehot.astype(jnp.float32), axis=0, keepdims=True))
    counts = jnp.concatenate(count_chunks, axis=1)

    zq_ref[...] = z + (zq - z)
    diff = zq - z
    partial = jnp.sum(diff * diff)

    @pl.when(i == 0)
    def _init():
        counts_ref[...] = jnp.zeros_like(counts_ref)
        loss_ref[...] = jnp.zeros_like(loss_ref)
        perp_ref[...] = jnp.zeros_like(perp_ref)

    counts_ref[...] += counts
    loss_ref[...] += partial.reshape(1, 1)

    @pl.when(i == nprog - 1)
    def _finish():
        total_rows = jnp.float32(nprog * rows)
        m = loss_ref[...] / (total_rows * _CODE_DIM)
        loss_ref[...] = m + _BETA * m
        avg = counts_ref[...] / total_rows
        ent = jnp.sum(avg * jnp.log(avg + 1e-10))
        perp_ref[...] = jnp.exp(-ent).reshape(1, 1)


@functools.partial(jax.jit, static_argnames=())
def kernel(z_e, codebook):
    B, K, C = z_e.shape
    n_rows = B * K
    z = z_e.reshape(n_rows, C)
    z2 = jnp.sum(z ** 2, axis=1, keepdims=True)
    e2 = jnp.sum(codebook ** 2, axis=1)[None, :]

    grid = n_rows // _ROWS_PER_BLOCK
    zq, codes, _counts, loss, perp = pl.pallas_call(
        _vq_kernel,
        grid=(grid,),
        in_specs=[
            pl.BlockSpec((_ROWS_PER_BLOCK, C), lambda i: (i, 0)),
            pl.BlockSpec((_ROWS_PER_BLOCK, 1), lambda i: (i, 0)),
            pl.BlockSpec((1, _N_CODES), lambda i: (0, 0)),
            pl.BlockSpec((_N_CODES, C), lambda i: (0, 0)),
        ],
        out_specs=[
            pl.BlockSpec((_ROWS_PER_BLOCK, C), lambda i: (i, 0)),
            pl.BlockSpec((_ROWS_PER_BLOCK, 1), lambda i: (i, 0)),
            pl.BlockSpec((1, _N_CODES), lambda i: (0, 0)),
            pl.BlockSpec((1, 1), lambda i: (0, 0)),
            pl.BlockSpec((1, 1), lambda i: (0, 0)),
        ],
        out_shape=[
            jax.ShapeDtypeStruct((n_rows, C), jnp.float32),
            jax.ShapeDtypeStruct((n_rows, 1), jnp.int32),
            jax.ShapeDtypeStruct((1, _N_CODES), jnp.float32),
            jax.ShapeDtypeStruct((1, 1), jnp.float32),
            jax.ShapeDtypeStruct((1, 1), jnp.float32),
        ],
    )(z, z2, e2, codebook)

    z_q_st = zq.reshape(B, K, C)
    codes_out = codes.reshape(B, K)
    return (z_q_st, codes_out, loss.reshape(()), perp.reshape(()))


# speedup vs baseline: 1.5144x; 1.5144x over previous
import functools

import jax
import jax.numpy as jnp
from jax import lax
from jax.experimental import pallas as pl
from jax.experimental.pallas import tpu as pltpu
from jax.experimental.pallas import tpu_sc as plsc

_N_CODES = 8192
_CODE_DIM = 32
_BETA = 0.25
_ROWS_PER_BLOCK = 1024
_ARG_CHUNK = 2048


def _vq_tc_kernel(z_ref, z2_ref, e2_ref, cb_ref,
                  codes_ref, counts_ref, loss_ref, perp_ref):
    i = pl.program_id(0)
    nprog = pl.num_programs(0)
    z = z_ref[...]
    z2 = z2_ref[...]
    e2 = e2_ref[...]
    rows = z.shape[0]

    run_min = jnp.full((rows, 1), jnp.inf, dtype=jnp.float32)
    run_val = jnp.zeros((rows, 1), dtype=jnp.float32)
    run_idx = jnp.zeros((rows, 1), dtype=jnp.int32)
    lanes = lax.broadcasted_iota(jnp.int32, (rows, _ARG_CHUNK), 1)

    zb = z.astype(jnp.bfloat16)
    for j in range(_N_CODES // _ARG_CHUNK):
        cbj = cb_ref[j * _ARG_CHUNK:(j + 1) * _ARG_CHUNK, :]
        ze = lax.dot_general(zb, cbj.astype(jnp.bfloat16),
                             (((1,), (1,)), ((), ())),
                             preferred_element_type=jnp.float32)
        e2j = e2[:, j * _ARG_CHUNK:(j + 1) * _ARG_CHUNK]
        dist = (z2 + e2j) - 2.0 * ze
        cmin = jnp.min(dist, axis=1, keepdims=True)
        cidx = jnp.min(jnp.where(dist == cmin, lanes, jnp.int32(2**30)),
                       axis=1, keepdims=True) + j * _ARG_CHUNK
        upd = cmin < run_min
        run_idx = jnp.where(upd, cidx, run_idx)
        run_val = jnp.where(upd, cmin, run_val)
        run_min = jnp.where(
            upd, cmin.astype(jnp.bfloat16).astype(jnp.float32), run_min)

    codes_ref[...] = run_idx

    count_chunks = []
    for j in range(_N_CODES // _ARG_CHUNK):
        onehot = (run_idx == lanes + j * _ARG_CHUNK).astype(jnp.float32)
        count_chunks.append(jnp.sum(onehot, axis=0, keepdims=True))
    counts = jnp.concatenate(count_chunks, axis=1)
    partial = jnp.sum(jnp.maximum(run_val, 0.0))

    @pl.when(i == 0)
    def _init():
        counts_ref[...] = jnp.zeros_like(counts_ref)
        loss_ref[...] = jnp.zeros_like(loss_ref)
        perp_ref[...] = jnp.zeros_like(perp_ref)

    counts_ref[...] += counts
    loss_ref[...] += partial.reshape(1, 1)

    @pl.when(i == nprog - 1)
    def _finish():
        total_rows = jnp.float32(nprog * rows)
        m = loss_ref[...] / (total_rows * _CODE_DIM)
        loss_ref[...] = m + _BETA * m
        avg = counts_ref[...] / total_rows
        ent = jnp.sum(avg * jnp.log(avg + 1e-10))
        perp_ref[...] = jnp.exp(-ent).reshape(1, 1)


def _make_sc_gather(n_rows, dim):
    info = plsc.get_sparse_core_info()
    n_workers = info.num_cores * info.num_subcores
    b_per_w = n_rows // n_workers
    mesh = plsc.VectorSubcoreMesh(core_axis_name="c", subcore_axis_name="s")
    n_sub = (b_per_w + 127) // 128
    sub = b_per_w // n_sub

    @functools.partial(
        pl.kernel, mesh=mesh,
        out_type=jax.ShapeDtypeStruct((n_rows, dim), jnp.float32),
        scratch_types=[
            pltpu.VMEM((b_per_w,), jnp.int32),
            pltpu.VMEM((b_per_w, dim), jnp.float32),
            pltpu.SemaphoreType.DMA,
        ],
    )
    def gather_kernel(table_hbm, idx_hbm, out_hbm, idx_v, rows_v, sem):
        wid = (lax.axis_index("s") * info.num_cores + lax.axis_index("c"))
        base = wid * b_per_w
        pltpu.sync_copy(idx_hbm.at[pl.ds(base, b_per_w)], idx_v)
        for k in range(n_sub):
            pltpu.async_copy(table_hbm.at[idx_v.at[pl.ds(k * sub, sub)]],
                             rows_v.at[pl.ds(k * sub, sub)], sem).wait()
        pltpu.sync_copy(rows_v, out_hbm.at[pl.ds(base, b_per_w)])

    return gather_kernel


def kernel(z_e, codebook):
    B, K, C = z_e.shape
    n_rows = B * K
    z = z_e.reshape(n_rows, C)
    z2 = jnp.sum(z ** 2, axis=1, keepdims=True)
    e2 = jnp.sum(codebook ** 2, axis=1)[None, :]

    grid = n_rows // _ROWS_PER_BLOCK
    codes, _counts, loss, perp = pl.pallas_call(
        _vq_tc_kernel,
        grid=(grid,),
        in_specs=[
            pl.BlockSpec((_ROWS_PER_BLOCK, C), lambda i: (i, 0)),
            pl.BlockSpec((_ROWS_PER_BLOCK, 1), lambda i: (i, 0)),
            pl.BlockSpec((1, _N_CODES), lambda i: (0, 0)),
            pl.BlockSpec((_N_CODES, C), lambda i: (0, 0)),
        ],
        out_specs=[
            pl.BlockSpec((_ROWS_PER_BLOCK, 1), lambda i: (i, 0)),
            pl.BlockSpec((1, _N_CODES), lambda i: (0, 0)),
            pl.BlockSpec((1, 1), lambda i: (0, 0)),
            pl.BlockSpec((1, 1), lambda i: (0, 0)),
        ],
        out_shape=[
            jax.ShapeDtypeStruct((n_rows, 1), jnp.int32),
            jax.ShapeDtypeStruct((1, _N_CODES), jnp.float32),
            jax.ShapeDtypeStruct((1, 1), jnp.float32),
            jax.ShapeDtypeStruct((1, 1), jnp.float32),
        ],
    )(z, z2, e2, codebook)

    idx_flat = codes.reshape(n_rows)
    cb_pad = jnp.pad(codebook, ((0, 0), (0, 128 - C)))
    zq = _make_sc_gather(n_rows, 128)(cb_pad, idx_flat)[:, :C]

    z_q_st = zq.reshape(B, K, C)
    codes_out = codes.reshape(B, K)
    return (z_q_st, codes_out, loss.reshape(()), perp.reshape(()))
